# baseline (speedup 1.0000x reference)
import functools

import jax
import jax.numpy as jnp
from jax import lax
from jax.experimental import pallas as pl
from jax.experimental.pallas import tpu as pltpu
from jax.experimental.pallas import tpu_sc as plsc

N = 10000
E = 320000
NFEAT = 128
HID = 16
NCLASS = 40

NC = 2
NS = 16
NW = NC * NS
B = 128
EB = E // B
NBF = EB // NW
NXTRA = EB - NBF * NW
NBT = NBF + 1
NSLOT = 24
AHEAD = 12
N_PAD = 10240
RPT = N_PAD // NS
VPC = N_PAD * HID // 128
N128 = N * HID // 128
assert EB * B == E and NXTRA <= NW


def _guard_call(guard, fn):
    if guard is None:
        fn()
    else:
        pl.when(guard)(fn)


_mesh = plsc.VectorSubcoreMesh(
    core_axis_name="c", subcore_axis_name="s", num_cores=NC)
_sc_params = pltpu.CompilerParams(use_tc_tiling_on_sc=False)


def _sc_copyout(acc, bounce, out_hbm, c, s):
    pltpu.sync_copy(acc.at[pl.ds(s * RPT, RPT)], bounce)
    pltpu.sync_copy(bounce, out_hbm.at[pl.ds(c * N_PAD + s * RPT, RPT)])


def _sc_zero_acc(acc, bounce, s):
    zero = jnp.zeros((16,), jnp.float32)

    def zb(i, carry):
        for k in range(8):
            bounce[i * 8 + k] = zero
        return carry
    lax.fori_loop(0, RPT // 8, zb, 0)
    pltpu.sync_copy(bounce, acc.at[pl.ds(s * RPT, RPT)])


def _sc_load_slab(e_hbm, plane, idx, wid, extra):
    pltpu.sync_copy(e_hbm.at[plane, pl.ds(wid * NBF, NBF)],
                    idx.at[pl.ds(0, NBF)])
    pl.when(extra)(lambda: pltpu.sync_copy(
        e_hbm.at[plane, pl.ds(NBF * NW + wid, 1)], idx.at[pl.ds(NBF, 1)]))


@functools.partial(
    pl.kernel,
    out_type=jax.ShapeDtypeStruct((NC * N_PAD, HID), jnp.float32),
    mesh=_mesh,
    scratch_types=[
        pltpu.VMEM((NBT, B), jnp.int32),
        pltpu.VMEM((B, HID), jnp.float32),
        pltpu.VMEM((RPT, HID), jnp.float32),
        pltpu.VMEM_SHARED((N_PAD, HID), jnp.float32),
    ] + [pltpu.SemaphoreType.DMA] * NSLOT,
    compiler_params=_sc_params,
)
def _sc_degree16(e_hbm, out_hbm, cidx, ones_v, bounce, acc, *sems):
    c = lax.axis_index("c")
    s = lax.axis_index("s")
    wid = c * NS + s
    extra = wid < NXTRA

    one = jnp.ones((16,), jnp.float32)

    def ob(i, carry):
        for k in range(4):
            ones_v[i * 4 + k] = one
        return carry
    lax.fori_loop(0, B // 4, ob, 0)
    _sc_zero_acc(acc, bounce, s)
    _sc_load_slab(e_hbm, 1, cidx, wid, extra)
    plsc.subcore_barrier()

    sh = [None] * NSLOT
    for j in range(NBT):
        sl = j % NSLOT
        guard = extra if j == NBF else None
        if sh[sl] is not None:
            h, g = sh[sl]
            _guard_call(g, h.wait)

        def fire(sl=sl, j=j, guard=guard):
            sh[sl] = (pltpu.async_copy(ones_v, acc.at[cidx.at[j]],
                                       sems[sl], add=True), guard)
        _guard_call(guard, fire)
    for sl in range(NSLOT):
        if sh[sl] is not None:
            h, g = sh[sl]
            _guard_call(g, h.wait)

    plsc.subcore_barrier()
    _sc_copyout(acc, bounce, out_hbm, c, s)


@functools.partial(
    pl.kernel,
    out_type=jax.ShapeDtypeStruct((NC * N_PAD, HID), jnp.float32),
    mesh=_mesh,
    scratch_types=[
        pltpu.VMEM((NBT, B), jnp.int32),
        pltpu.VMEM((NBT, B), jnp.int32),
        pltpu.VMEM((NSLOT * B, HID), jnp.float32),
        pltpu.VMEM((RPT, HID), jnp.float32),
        pltpu.VMEM_SHARED((N_PAD, HID), jnp.float32),
    ] + [pltpu.SemaphoreType.DMA] * NSLOT,
    compiler_params=_sc_params,
)
def _sc_scatter16(g_hbm, e_hbm, out_hbm, ridx, cidx, rows, bounce, acc, *sems):
    c = lax.axis_index("c")
    s = lax.axis_index("s")
    wid = c * NS + s
    extra = wid < NXTRA

    _sc_zero_acc(acc, bounce, s)
    _sc_load_slab(e_hbm, 0, ridx, wid, extra)
    _sc_load_slab(e_hbm, 1, cidx, wid, extra)
    plsc.subcore_barrier()

    gh = [None] * NSLOT
    sh = [None] * NSLOT

    def fire_gather(j):
        sl = j % NSLOT
        guard = extra if j == NBF else None
        if sh[sl] is not None:
            h, g = sh[sl]
            _guard_call(g, h.wait)
            sh[sl] = None

        def fire(sl=sl, j=j, guard=guard):
            gh[sl] = (pltpu.async_copy(g_hbm.at[ridx.at[j]],
                                       rows.at[pl.ds(sl * B, B)],
                                       sems[sl]), guard)
        _guard_call(guard, fire)

    for j in range(min(AHEAD, NBT)):
        fire_gather(j)
    for j in range(NBT):
        sl = j % NSLOT
        guard = extra if j == NBF else None
        if j + AHEAD < NBT:
            fire_gather(j + AHEAD)

        def consume(sl=sl, j=j, guard=guard):
            gh[sl][0].wait()
            sh[sl] = (pltpu.async_copy(rows.at[pl.ds(sl * B, B)],
                                       acc.at[cidx.at[j]],
                                       sems[sl], add=True), guard)
        _guard_call(guard, consume)
    for sl in range(NSLOT):
        if sh[sl] is not None:
            h, g = sh[sl]
            _guard_call(g, h.wait)

    plsc.subcore_barrier()
    _sc_copyout(acc, bounce, out_hbm, c, s)


def _tc_head(degv_ref, xv_ref, w1b_ref, g1_ref, disrep_ref):
    deg = degv_ref[0, 0:N128, :] + degv_ref[1, 0:N128, :] + 1.0
    disrep = lax.rsqrt(deg)
    h1 = jnp.dot(xv_ref[...], w1b_ref[...],
                 preferred_element_type=jnp.float32)
    disrep_ref[...] = disrep
    g1_ref[...] = h1 * disrep


def _tc_mid(pv_ref, g1_ref, disrep_ref, b1r_ref, g2_ref):
    disrep = disrep_ref[...]
    s1 = disrep * (pv_ref[0, 0:N128, :] + pv_ref[1, 0:N128, :] + g1_ref[...])
    a1 = jnp.maximum(s1 + b1r_ref[...], 0.0)
    g2_ref[...] = a1 * disrep


def _tc_tail(pv_ref, g2_ref, disrep_ref, w2b_ref, b2r_ref, out_ref):
    s2 = disrep_ref[...] * (
        pv_ref[0, 0:N128, :] + pv_ref[1, 0:N128, :] + g2_ref[...])
    logits = jnp.dot(s2, w2b_ref[...],
                     preferred_element_type=jnp.float32) + b2r_ref[...]
    for k in range(8):
        seg = logits[:, 40 * k:40 * k + 40]
        m = jnp.max(seg, axis=1, keepdims=True)
        lse = jnp.log(jnp.sum(jnp.exp(seg - m), axis=1, keepdims=True)) + m
        out_ref[:, 40 * k:40 * k + 40] = seg - lse


def kernel(x, edge_index, W1, b1, W2, b2):
    e3 = edge_index.astype(jnp.int32).reshape(2, EB, B)
    w1b = jnp.kron(jnp.eye(8, dtype=jnp.float32), W1)

    degrep = _sc_degree16(e3)
    degv = degrep.reshape(NC, VPC, 128)
    xv = x.reshape(N128, 8 * NFEAT)

    g1_128, disrep = pl.pallas_call(
        _tc_head,
        out_shape=(jax.ShapeDtypeStruct((N128, 128), jnp.float32),
                   jax.ShapeDtypeStruct((N128, 128), jnp.float32)),
    )(degv, xv, w1b)

    p1 = _sc_scatter16(g1_128.reshape(N, HID), e3)
    g2_128 = pl.pallas_call(
        _tc_mid,
        out_shape=jax.ShapeDtypeStruct((N128, 128), jnp.float32),
    )(p1.reshape(NC, VPC, 128), g1_128, disrep, jnp.tile(b1, 8).reshape(1, 128))

    p2 = _sc_scatter16(g2_128.reshape(N, HID), e3)
    w2b = jnp.kron(jnp.eye(8, dtype=jnp.float32), W2)
    out320 = pl.pallas_call(
        _tc_tail,
        out_shape=jax.ShapeDtypeStruct((N128, 8 * NCLASS), jnp.float32),
    )(p2.reshape(NC, VPC, 128), g2_128, disrep, w2b,
      jnp.tile(b2, 8).reshape(1, 8 * NCLASS))
    return out320.reshape(N, NCLASS)


# speedup vs baseline: 1.0236x; 1.0236x over previous
import functools

import jax
import jax.numpy as jnp
from jax import lax
from jax.experimental import pallas as pl
from jax.experimental.pallas import tpu as pltpu
from jax.experimental.pallas import tpu_sc as plsc

N = 10000
E = 320000
NFEAT = 128
HID = 16
NCLASS = 40

NC = 2
NS = 16
NW = NC * NS
B = 128
EB = E // B
NBF = EB // NW
NXTRA = EB - NBF * NW
NBT = NBF + 1
NSLOT = 16
AHEAD = 8
N_PAD = 10240
RPT = N_PAD // NS
VPC = N_PAD * HID // 128
N128 = N * HID // 128
assert EB * B == E and NXTRA <= NW


def _guard_call(guard, fn):
    if guard is None:
        fn()
    else:
        pl.when(guard)(fn)


_mesh = plsc.VectorSubcoreMesh(
    core_axis_name="c", subcore_axis_name="s", num_cores=NC)
_sc_params = pltpu.CompilerParams(use_tc_tiling_on_sc=False)


def _sc_copyout(acc, bounce, out_hbm, c, s):
    pltpu.sync_copy(acc.at[pl.ds(s * RPT, RPT)], bounce)
    pltpu.sync_copy(bounce, out_hbm.at[pl.ds(c * N_PAD + s * RPT, RPT)])


def _sc_zero_acc(acc, bounce, s):
    zero = jnp.zeros((16,), jnp.float32)

    def zb(i, carry):
        for k in range(8):
            bounce[i * 8 + k] = zero
        return carry
    lax.fori_loop(0, RPT // 8, zb, 0)
    pltpu.sync_copy(bounce, acc.at[pl.ds(s * RPT, RPT)])


def _sc_load_slab(e_hbm, plane, idx, wid, extra):
    pltpu.sync_copy(e_hbm.at[plane, pl.ds(wid * NBF, NBF)],
                    idx.at[pl.ds(0, NBF)])
    pl.when(extra)(lambda: pltpu.sync_copy(
        e_hbm.at[plane, pl.ds(NBF * NW + wid, 1)], idx.at[pl.ds(NBF, 1)]))


@functools.partial(
    pl.kernel,
    out_type=jax.ShapeDtypeStruct((NC * N_PAD,), jnp.float32),
    mesh=_mesh,
    scratch_types=[
        pltpu.VMEM((NBT, B), jnp.int32),
        pltpu.VMEM((B,), jnp.float32),
        pltpu.VMEM((RPT,), jnp.float32),
        pltpu.VMEM_SHARED((N_PAD,), jnp.float32),
    ] + [pltpu.SemaphoreType.DMA] * NSLOT,
    compiler_params=_sc_params,
)
def _sc_degree(e_hbm, out_hbm, cidx, ones_v, bounce, acc, *sems):
    c = lax.axis_index("c")
    s = lax.axis_index("s")
    wid = c * NS + s
    extra = wid < NXTRA

    one = jnp.ones((16,), jnp.float32)
    zero = jnp.zeros((16,), jnp.float32)
    for i in range(B // 16):
        ones_v[pl.ds(i * 16, 16)] = one

    def zb(i, carry):
        for k in range(4):
            bounce[pl.ds((i * 4 + k) * 16, 16)] = zero
        return carry
    lax.fori_loop(0, RPT // 64, zb, 0)
    pltpu.sync_copy(bounce, acc.at[pl.ds(s * RPT, RPT)])
    _sc_load_slab(e_hbm, 1, cidx, wid, extra)
    plsc.subcore_barrier()

    sh = [None] * NSLOT
    for j in range(NBT):
        sl = j % NSLOT
        guard = extra if j == NBF else None
        if sh[sl] is not None:
            h, g = sh[sl]
            _guard_call(g, h.wait)

        def fire(sl=sl, j=j, guard=guard):
            sh[sl] = (pltpu.async_copy(ones_v, acc.at[cidx.at[j]],
                                       sems[sl], add=True), guard)
        _guard_call(guard, fire)
    for sl in range(NSLOT):
        if sh[sl] is not None:
            h, g = sh[sl]
            _guard_call(g, h.wait)

    plsc.subcore_barrier()
    pltpu.sync_copy(acc.at[pl.ds(s * RPT, RPT)], bounce)
    pltpu.sync_copy(bounce, out_hbm.at[pl.ds(c * N_PAD + s * RPT, RPT)])


@functools.partial(
    pl.kernel,
    out_type=jax.ShapeDtypeStruct((NC * N_PAD, HID), jnp.float32),
    mesh=_mesh,
    scratch_types=[
        pltpu.VMEM((NBT, B), jnp.int32),
        pltpu.VMEM((NBT, B), jnp.int32),
        pltpu.VMEM((NSLOT * B, HID), jnp.float32),
        pltpu.VMEM((RPT, HID), jnp.float32),
        pltpu.VMEM_SHARED((N_PAD, HID), jnp.float32),
    ] + [pltpu.SemaphoreType.DMA] * NSLOT,
    compiler_params=_sc_params,
)
def _sc_scatter16(g_hbm, e_hbm, out_hbm, ridx, cidx, rows, bounce, acc, *sems):
    c = lax.axis_index("c")
    s = lax.axis_index("s")
    wid = c * NS + s
    extra = wid < NXTRA

    _sc_zero_acc(acc, bounce, s)
    _sc_load_slab(e_hbm, 0, ridx, wid, extra)
    _sc_load_slab(e_hbm, 1, cidx, wid, extra)
    plsc.subcore_barrier()

    gh = [None] * NSLOT
    sh = [None] * NSLOT

    def fire_gather(j):
        sl = j % NSLOT
        guard = extra if j == NBF else None
        if sh[sl] is not None:
            h, g = sh[sl]
            _guard_call(g, h.wait)
            sh[sl] = None

        def fire(sl=sl, j=j, guard=guard):
            gh[sl] = (pltpu.async_copy(g_hbm.at[ridx.at[j]],
                                       rows.at[pl.ds(sl * B, B)],
                                       sems[sl]), guard)
        _guard_call(guard, fire)

    for j in range(min(AHEAD, NBT)):
        fire_gather(j)
    for j in range(NBT):
        sl = j % NSLOT
        guard = extra if j == NBF else None
        if j + AHEAD < NBT:
            fire_gather(j + AHEAD)

        def consume(sl=sl, j=j, guard=guard):
            gh[sl][0].wait()
            sh[sl] = (pltpu.async_copy(rows.at[pl.ds(sl * B, B)],
                                       acc.at[cidx.at[j]],
                                       sems[sl], add=True), guard)
        _guard_call(guard, consume)
    for sl in range(NSLOT):
        if sh[sl] is not None:
            h, g = sh[sl]
            _guard_call(g, h.wait)

    plsc.subcore_barrier()
    _sc_copyout(acc, bounce, out_hbm, c, s)


def _tc_head(degv_ref, pmat_ref, xv_ref, w1b_ref, g1_ref, disrep_ref):
    deg8 = degv_ref[0, 0:N128, :] + degv_ref[1, 0:N128, :] + 1.0
    dis8 = lax.rsqrt(deg8)
    disrep = jnp.dot(dis8, pmat_ref[...], preferred_element_type=jnp.float32)
    h1 = jnp.dot(xv_ref[...], w1b_ref[...],
                 preferred_element_type=jnp.float32)
    disrep_ref[...] = disrep
    g1_ref[...] = h1 * disrep


def _tc_mid(pv_ref, g1_ref, disrep_ref, b1r_ref, g2_ref):
    disrep = disrep_ref[...]
    s1 = disrep * (pv_ref[0, 0:N128, :] + pv_ref[1, 0:N128, :] + g1_ref[...])
    a1 = jnp.maximum(s1 + b1r_ref[...], 0.0)
    g2_ref[...] = a1 * disrep


def _tc_tail(pv_ref, g2_ref, disrep_ref, w2b_ref, b2r_ref, out_ref):
    s2 = disrep_ref[...] * (
        pv_ref[0, 0:N128, :] + pv_ref[1, 0:N128, :] + g2_ref[...])
    logits = jnp.dot(s2, w2b_ref[...],
                     preferred_element_type=jnp.float32) + b2r_ref[...]
    for k in range(8):
        seg = logits[:, 40 * k:40 * k + 40]
        m = jnp.max(seg, axis=1, keepdims=True)
        lse = jnp.log(jnp.sum(jnp.exp(seg - m), axis=1, keepdims=True)) + m
        out_ref[k::8, :] = seg - lse


def kernel(x, edge_index, W1, b1, W2, b2):
    e3 = edge_index.astype(jnp.int32).reshape(2, EB, B)
    w1b = jnp.kron(jnp.eye(8, dtype=jnp.float32), W1)

    deg = _sc_degree(e3)
    degv = deg.reshape(NC, N_PAD // 8, 8)
    pmat = (jnp.arange(128, dtype=jnp.int32)[None, :] // 16
            == jnp.arange(8, dtype=jnp.int32)[:, None]).astype(jnp.float32)
    xv = x.reshape(N128, 8 * NFEAT)

    g1_128, disrep = pl.pallas_call(
        _tc_head,
        out_shape=(jax.ShapeDtypeStruct((N128, 128), jnp.float32),
                   jax.ShapeDtypeStruct((N128, 128), jnp.float32)),
    )(degv, pmat, xv, w1b)

    p1 = _sc_scatter16(g1_128.reshape(N, HID), e3)
    g2_128 = pl.pallas_call(
        _tc_mid,
        out_shape=jax.ShapeDtypeStruct((N128, 128), jnp.float32),
    )(p1.reshape(NC, VPC, 128), g1_128, disrep, jnp.tile(b1, 8).reshape(1, 128))

    p2 = _sc_scatter16(g2_128.reshape(N, HID), e3)
    w2b = jnp.kron(jnp.eye(8, dtype=jnp.float32), W2)
    out = pl.pallas_call(
        _tc_tail,
        out_shape=jax.ShapeDtypeStruct((N, NCLASS), jnp.float32),
    )(p2.reshape(NC, VPC, 128), g2_128, disrep, w2b,
      jnp.tile(b2, 8).reshape(1, 8 * NCLASS))
    return out


# speedup vs baseline: 1.0517x; 1.0275x over previous
import functools

import jax
import jax.numpy as jnp
from jax import lax
from jax.experimental import pallas as pl
from jax.experimental.pallas import tpu as pltpu
from jax.experimental.pallas import tpu_sc as plsc

N = 10000
E = 320000
NFEAT = 128
HID = 16
NCLASS = 40

NC = 2
NS = 16
NW = NC * NS
B = 128
EB = E // B
NBF = EB // NW
NXTRA = EB - NBF * NW
NBT = NBF + 1
NSLOT = 16
AHEAD = 8
N_PAD = 10240
RPT = N_PAD // NS
VPC = N_PAD * HID // 128
N128 = N * HID // 128
assert EB * B == E and NXTRA <= NW


def _guard_call(guard, fn):
    if guard is None:
        fn()
    else:
        pl.when(guard)(fn)


_mesh = plsc.VectorSubcoreMesh(
    core_axis_name="c", subcore_axis_name="s", num_cores=NC)
_sc_params = pltpu.CompilerParams(use_tc_tiling_on_sc=False)


def _sc_copyout(acc, bounce, out_hbm, c, s):
    pltpu.sync_copy(acc.at[pl.ds(s * RPT, RPT)], bounce)
    pltpu.sync_copy(bounce, out_hbm.at[pl.ds(c * N_PAD + s * RPT, RPT)])


def _sc_zero_acc(acc, bounce, s):
    zero = jnp.zeros((16,), jnp.float32)

    def zb(i, carry):
        for k in range(8):
            bounce[i * 8 + k] = zero
        return carry
    lax.fori_loop(0, RPT // 8, zb, 0)
    pltpu.sync_copy(bounce, acc.at[pl.ds(s * RPT, RPT)])


def _sc_load_slab(e_hbm, plane, idx, wid, extra):
    pltpu.sync_copy(e_hbm.at[plane, pl.ds(wid * NBF, NBF)],
                    idx.at[pl.ds(0, NBF)])
    pl.when(extra)(lambda: pltpu.sync_copy(
        e_hbm.at[plane, pl.ds(NBF * NW + wid, 1)], idx.at[pl.ds(NBF, 1)]))


@functools.partial(
    pl.kernel,
    out_type=jax.ShapeDtypeStruct((NC * N_PAD,), jnp.float32),
    mesh=_mesh,
    scratch_types=[
        pltpu.VMEM((NBT, B), jnp.int32),
        pltpu.VMEM((B,), jnp.float32),
        pltpu.VMEM((RPT,), jnp.float32),
        pltpu.VMEM_SHARED((N_PAD,), jnp.float32),
    ] + [pltpu.SemaphoreType.DMA] * NSLOT,
    compiler_params=_sc_params,
)
def _sc_degree(e_hbm, out_hbm, cidx, ones_v, bounce, acc, *sems):
    c = lax.axis_index("c")
    s = lax.axis_index("s")
    wid = c * NS + s
    extra = wid < NXTRA

    one = jnp.ones((16,), jnp.float32)
    zero = jnp.zeros((16,), jnp.float32)
    for i in range(B // 16):
        ones_v[pl.ds(i * 16, 16)] = one

    def zb(i, carry):
        for k in range(4):
            bounce[pl.ds((i * 4 + k) * 16, 16)] = zero
        return carry
    lax.fori_loop(0, RPT // 64, zb, 0)
    pltpu.sync_copy(bounce, acc.at[pl.ds(s * RPT, RPT)])
    _sc_load_slab(e_hbm, 1, cidx, wid, extra)
    plsc.subcore_barrier()

    sh = [None] * NSLOT
    for j in range(NBT):
        sl = j % NSLOT
        guard = extra if j == NBF else None
        if sh[sl] is not None:
            h, g = sh[sl]
            _guard_call(g, h.wait)

        def fire(sl=sl, j=j, guard=guard):
            sh[sl] = (pltpu.async_copy(ones_v, acc.at[cidx.at[j]],
                                       sems[sl], add=True), guard)
        _guard_call(guard, fire)
    for sl in range(NSLOT):
        if sh[sl] is not None:
            h, g = sh[sl]
            _guard_call(g, h.wait)

    plsc.subcore_barrier()
    pltpu.sync_copy(acc.at[pl.ds(s * RPT, RPT)], bounce)
    pltpu.sync_copy(bounce, out_hbm.at[pl.ds(c * N_PAD + s * RPT, RPT)])


@functools.partial(
    pl.kernel,
    out_type=jax.ShapeDtypeStruct((NC * N_PAD, HID), jnp.float32),
    mesh=_mesh,
    scratch_types=[
        pltpu.VMEM((NBT, B), jnp.int32),
        pltpu.VMEM((NBT, B), jnp.int32),
        pltpu.VMEM((NSLOT * B, HID), jnp.float32),
        pltpu.VMEM((RPT, HID), jnp.float32),
        pltpu.VMEM_SHARED((N_PAD, HID), jnp.float32),
    ] + [pltpu.SemaphoreType.DMA] * NSLOT,
    compiler_params=_sc_params,
)
def _sc_scatter16(g_hbm, e_hbm, out_hbm, ridx, cidx, rows, bounce, acc, *sems):
    c = lax.axis_index("c")
    s = lax.axis_index("s")
    wid = c * NS + s
    extra = wid < NXTRA

    _sc_zero_acc(acc, bounce, s)
    _sc_load_slab(e_hbm, 0, ridx, wid, extra)
    _sc_load_slab(e_hbm, 1, cidx, wid, extra)
    plsc.subcore_barrier()

    gh = [None] * NSLOT
    sh = [None] * NSLOT

    def fire_gather(j):
        sl = j % NSLOT
        guard = extra if j == NBF else None
        if sh[sl] is not None:
            h, g = sh[sl]
            _guard_call(g, h.wait)
            sh[sl] = None

        def fire(sl=sl, j=j, guard=guard):
            gh[sl] = (pltpu.async_copy(g_hbm.at[ridx.at[j]],
                                       rows.at[pl.ds(sl * B, B)],
                                       sems[sl]), guard)
        _guard_call(guard, fire)

    for j in range(min(AHEAD, NBT)):
        fire_gather(j)
    for j in range(NBT):
        sl = j % NSLOT
        guard = extra if j == NBF else None
        if j + AHEAD < NBT:
            fire_gather(j + AHEAD)

        def consume(sl=sl, j=j, guard=guard):
            gh[sl][0].wait()
            sh[sl] = (pltpu.async_copy(rows.at[pl.ds(sl * B, B)],
                                       acc.at[cidx.at[j]],
                                       sems[sl], add=True), guard)
        _guard_call(guard, consume)
    for sl in range(NSLOT):
        if sh[sl] is not None:
            h, g = sh[sl]
            _guard_call(g, h.wait)

    plsc.subcore_barrier()
    _sc_copyout(acc, bounce, out_hbm, c, s)


def _tc_head(degv_ref, pmat_ref, xv_ref, w1b_ref, g1_ref, disrep_ref):
    deg8 = degv_ref[0, 0:N128, :] + degv_ref[1, 0:N128, :] + 1.0
    dis8 = lax.rsqrt(deg8)
    disrep = jnp.dot(dis8, pmat_ref[...], preferred_element_type=jnp.float32)
    h1 = jnp.dot(xv_ref[...], w1b_ref[...],
                 preferred_element_type=jnp.float32)
    disrep_ref[...] = disrep
    g1_ref[...] = h1 * disrep


def _tc_mid(pv_ref, g1_ref, disrep_ref, b1r_ref, g2_ref):
    disrep = disrep_ref[...]
    s1 = disrep * (pv_ref[0, 0:N128, :] + pv_ref[1, 0:N128, :] + g1_ref[...])
    a1 = jnp.maximum(s1 + b1r_ref[...], 0.0)
    g2_ref[...] = a1 * disrep


def _tc_tail(pv_ref, g2_ref, disrep_ref, w2b_ref, b2r_ref, sel_ref, out_ref):
    s2 = disrep_ref[...] * (
        pv_ref[0, 0:N128, :] + pv_ref[1, 0:N128, :] + g2_ref[...])
    logits = jnp.dot(s2, w2b_ref[...],
                     preferred_element_type=jnp.float32) + b2r_ref[...]
    sel = sel_ref[...]
    m8 = jnp.concatenate(
        [jnp.max(logits[:, 40 * k:40 * k + 40], axis=1, keepdims=True)
         for k in range(8)], axis=1)
    m320 = jnp.dot(m8, sel, preferred_element_type=jnp.float32)
    s8 = jnp.dot(jnp.exp(logits - m320), sel.T,
                 preferred_element_type=jnp.float32)
    lse8 = jnp.log(s8) + m8
    res = logits - jnp.dot(lse8, sel, preferred_element_type=jnp.float32)
    for k in range(8):
        out_ref[k::8, :] = res[:, 40 * k:40 * k + 40]


def kernel(x, edge_index, W1, b1, W2, b2):
    e3 = edge_index.astype(jnp.int32).reshape(2, EB, B)
    w1b = jnp.kron(jnp.eye(8, dtype=jnp.float32), W1)

    deg = _sc_degree(e3)
    degv = deg.reshape(NC, N_PAD // 8, 8)
    pmat = (jnp.arange(128, dtype=jnp.int32)[None, :] // 16
            == jnp.arange(8, dtype=jnp.int32)[:, None]).astype(jnp.float32)
    xv = x.reshape(N128, 8 * NFEAT)

    g1_128, disrep = pl.pallas_call(
        _tc_head,
        out_shape=(jax.ShapeDtypeStruct((N128, 128), jnp.float32),
                   jax.ShapeDtypeStruct((N128, 128), jnp.float32)),
    )(degv, pmat, xv, w1b)

    p1 = _sc_scatter16(g1_128.reshape(N, HID), e3)
    g2_128 = pl.pallas_call(
        _tc_mid,
        out_shape=jax.ShapeDtypeStruct((N128, 128), jnp.float32),
    )(p1.reshape(NC, VPC, 128), g1_128, disrep, jnp.tile(b1, 8).reshape(1, 128))

    p2 = _sc_scatter16(g2_128.reshape(N, HID), e3)
    w2b = jnp.kron(jnp.eye(8, dtype=jnp.float32), W2)
    out = pl.pallas_call(
        _tc_tail,
        out_shape=jax.ShapeDtypeStruct((N, NCLASS), jnp.float32),
    )(p2.reshape(NC, VPC, 128), g2_128, disrep, w2b,
      jnp.tile(b2, 8).reshape(1, 8 * NCLASS),
      (jnp.arange(8 * NCLASS, dtype=jnp.int32)[None, :] // NCLASS
       == jnp.arange(8, dtype=jnp.int32)[:, None]).astype(jnp.float32))
    return out
